# baseline (speedup 1.0000x reference)
import jax
import jax.numpy as jnp
from jax.experimental import pallas as pl
from jax.experimental.pallas import tpu as pltpu


def _sigmoid(x):
    return 0.5 * jnp.tanh(0.5 * x) + 0.5


def _ffn_body(x_ref, w1_ref, w3_ref, w2_ref, wf_ref, dec_ref, wfb_ref):
    j = pl.program_id(0)
    x = x_ref[...]
    w1 = w1_ref[...].astype(jnp.bfloat16)
    w3 = w3_ref[...].astype(jnp.bfloat16)
    w2 = w2_ref[...].astype(jnp.bfloat16)
    a = jnp.dot(x, w1, preferred_element_type=jnp.float32)
    c = jnp.dot(x, w3, preferred_element_type=jnp.float32)
    h = ((a * _sigmoid(a)) * c).astype(jnp.bfloat16)
    part = jnp.dot(h, w2, preferred_element_type=jnp.float32)

    @pl.when(j == 0)
    def _init():
        dec_ref[...] = part

    @pl.when(j != 0)
    def _acc():
        dec_ref[...] += part

    wfb_ref[...] = wf_ref[...].astype(jnp.bfloat16)


def _out_body(dec_ref, rms_ref, wfb_ref, bias_ref, out_ref, d_ref):
    ob = pl.program_id(1)
    BO = out_ref.shape[1]

    @pl.when(ob == 0)
    def _norm():
        dec = dec_ref[...]
        dec = dec * _sigmoid(dec)
        dec = dec * jax.lax.rsqrt(
            jnp.mean(dec * dec, axis=-1, keepdims=True) + 1e-6)
        dec = dec * rms_ref[...]
        d_ref[...] = dec.astype(jnp.bfloat16)

    logit = jnp.dot(d_ref[...], wfb_ref[:, pl.ds(ob * BO, BO)],
                    preferred_element_type=jnp.float32)
    logit = logit + bias_ref[:, pl.ds(ob * BO, BO)]
    out_ref[...] = _sigmoid(logit)


@jax.jit
def _run(x, W1, W2, W3, rms_w, Wf, bf):
    L, D = x.shape
    F = W1.shape[1]
    O = Wf.shape[1]

    xb = x.astype(jnp.bfloat16)
    rms2 = rms_w.reshape(1, D)
    bf2 = bf.reshape(1, O)

    NJ = 16
    FC = F // NJ
    WFC = O // NJ

    dec, wfb = pl.pallas_call(
        _ffn_body,
        grid=(NJ,),
        in_specs=[
            pl.BlockSpec((L, D), lambda j: (0, 0)),
            pl.BlockSpec((D, FC), lambda j: (0, j)),
            pl.BlockSpec((D, FC), lambda j: (0, j)),
            pl.BlockSpec((FC, D), lambda j: (j, 0)),
            pl.BlockSpec((D, WFC), lambda j: (0, j)),
        ],
        out_specs=[
            pl.BlockSpec((L, D), lambda j: (0, 0)),
            pl.BlockSpec((D, WFC), lambda j: (0, j)),
        ],
        out_shape=[
            jax.ShapeDtypeStruct((L, D), jnp.float32),
            jax.ShapeDtypeStruct((D, O), jnp.bfloat16),
        ],
        compiler_params=pltpu.CompilerParams(
            dimension_semantics=("arbitrary",),
        ),
    )(xb, W1, W3, W2, Wf)

    BL_B = min(1024, L)
    BO_B = min(2048, O)
    out = pl.pallas_call(
        _out_body,
        grid=(L // BL_B, O // BO_B),
        in_specs=[
            pl.BlockSpec((BL_B, D), lambda lb, ob: (lb, 0)),
            pl.BlockSpec((1, D), lambda lb, ob: (0, 0)),
            pl.BlockSpec((D, O), lambda lb, ob: (0, 0)),
            pl.BlockSpec((1, O), lambda lb, ob: (0, 0)),
        ],
        out_specs=pl.BlockSpec((BL_B, BO_B), lambda lb, ob: (lb, ob)),
        out_shape=jax.ShapeDtypeStruct((L, O), jnp.float32),
        scratch_shapes=[pltpu.VMEM((BL_B, D), jnp.bfloat16)],
        compiler_params=pltpu.CompilerParams(
            dimension_semantics=("arbitrary", "arbitrary"),
        ),
    )(dec, rms2, wfb, bf2)
    return out


def kernel(inputs, label, W1, W2, W3, rms_w, Wf, bf):
    del label
    x = inputs[0]
    out = _run(x, W1, W2, W3, rms_w, Wf, bf)
    return out[None]
